# baseline (device time: 21097 ns/iter reference)
import jax
import jax.numpy as jnp
from jax import lax
from jax.experimental import pallas as pl
from jax.experimental.pallas import tpu as pltpu

N_DEV = 8
B, SQ, SKV, HL, DH = 2, 128, 128, 4, 64
DM = 512
DB = HL * DH

XOR_AXES = (1, 3, 4)
ROWS = B * SQ
CHUNKS = ((0, 88), (88, 88), (176, 80))


def kernel(x, Wq, K_ext, V_ext, Wo):
    idx = lax.axis_index("i")
    Wq_sl = lax.dynamic_slice_in_dim(Wq, idx * DB, DB, axis=1).astype(jnp.bfloat16)
    Wo_sl = lax.dynamic_slice_in_dim(Wo, idx * DB, DB, axis=0).astype(jnp.bfloat16)

    def body(x_ref, wq_ref, k_hbm, v_hbm, wo_ref, out_ref,
             kv_buf, send_ref, recv_ref, kv_sems, send_sems, recv_sems):
        my = lax.axis_index("i")
        nbrs = [lax.bitwise_xor(my, c) for c in XOR_AXES]

        kv_dmas = {}
        for b in range(B):
            for h in range(HL):
                for i, src in enumerate((k_hbm, v_hbm)):
                    dma = pltpu.make_async_copy(
                        src.at[b, :, h, :], kv_buf.at[i, b, h],
                        kv_sems.at[i, b, h])
                    dma.start()
                    kv_dmas[(i, b, h)] = dma

        barrier = pltpu.get_barrier_semaphore()
        for nbr in nbrs:
            pl.semaphore_signal(barrier, inc=1, device_id=(nbr,),
                                device_id_type=pl.DeviceIdType.MESH)

        xb = x_ref[...].reshape(ROWS, DM).astype(jnp.bfloat16)
        q = lax.dot(xb, wq_ref[...],
                    preferred_element_type=jnp.float32).astype(jnp.bfloat16)
        ctx_rows = []
        for b in range(B):
            head_cols = []
            for h in range(HL):
                qbh = q[b * SQ:(b + 1) * SQ, h * DH:(h + 1) * DH]
                kv_dmas[(0, b, h)].wait()
                kv_dmas[(1, b, h)].wait()
                kbh = kv_buf[0, b, h].astype(jnp.bfloat16)
                vbh = kv_buf[1, b, h].astype(jnp.bfloat16)
                s = lax.dot_general(
                    qbh, kbh, (((1,), (1,)), ((), ())),
                    preferred_element_type=jnp.float32) * 0.125
                w = jnp.exp(s)
                recip = 1.0 / jnp.sum(w, axis=1, keepdims=True)
                ctx = lax.dot(w.astype(jnp.bfloat16), vbh,
                              preferred_element_type=jnp.float32)
                head_cols.append((ctx * recip).astype(jnp.bfloat16))
            ctx_rows.append(jnp.concatenate(head_cols, axis=1))
        ctx_all = jnp.concatenate(ctx_rows, axis=0)
        acc = lax.dot(ctx_all, wo_ref[...],
                      preferred_element_type=jnp.float32)

        pl.semaphore_wait(barrier, len(nbrs))
        for s in range(3):
            rdmas = []
            for p, (r0, rn) in enumerate(CHUNKS):
                send_ref[s, pl.ds(r0, rn), :] = acc[r0:r0 + rn, :].astype(jnp.bfloat16)
                rdma = pltpu.make_async_remote_copy(
                    src_ref=send_ref.at[s, pl.ds(r0, rn)],
                    dst_ref=recv_ref.at[s, pl.ds(r0, rn)],
                    send_sem=send_sems.at[s, p],
                    recv_sem=recv_sems.at[s, p],
                    device_id=(nbrs[(s + p) % 3],),
                    device_id_type=pl.DeviceIdType.MESH,
                )
                rdma.start()
                rdmas.append(rdma)
            parts = []
            for p, (r0, rn) in enumerate(CHUNKS):
                rdmas[p].wait()
                parts.append(acc[r0:r0 + rn, :]
                             + recv_ref[s, pl.ds(r0, rn), :].astype(jnp.float32))
            acc = jnp.concatenate(parts, axis=0)
        out_ref[...] = acc.reshape(B, SQ, DM).astype(jnp.bfloat16)

    return pl.pallas_call(
        body,
        out_shape=jax.ShapeDtypeStruct((B, SQ, DM), jnp.bfloat16),
        in_specs=[
            pl.BlockSpec(memory_space=pltpu.VMEM),
            pl.BlockSpec(memory_space=pltpu.VMEM),
            pl.BlockSpec(memory_space=pl.ANY),
            pl.BlockSpec(memory_space=pl.ANY),
            pl.BlockSpec(memory_space=pltpu.VMEM),
        ],
        out_specs=pl.BlockSpec(memory_space=pltpu.VMEM),
        scratch_shapes=[
            pltpu.VMEM((2, B, HL, SKV, DH), jnp.float32),
            pltpu.VMEM((3, ROWS, DM), jnp.bfloat16),
            pltpu.VMEM((3, ROWS, DM), jnp.bfloat16),
            pltpu.SemaphoreType.DMA((2, B, HL)),
            pltpu.SemaphoreType.DMA((3, 3)),
            pltpu.SemaphoreType.DMA((3, 3)),
        ],
        compiler_params=pltpu.CompilerParams(collective_id=0),
    )(x, Wq_sl, K_ext, V_ext, Wo_sl)


# device time: 18542 ns/iter; 1.1378x vs baseline; 1.1378x over previous
import jax
import jax.numpy as jnp
from jax import lax
from jax.experimental import pallas as pl
from jax.experimental.pallas import tpu as pltpu

N_DEV = 8
B, SQ, SKV, HL, DH = 2, 128, 128, 4, 64
DM = 512
DB = HL * DH

XOR_AXES = (1, 3, 4)
ROWS = B * SQ
CHUNKS = ((0, 88), (88, 88), (176, 80))


def kernel(x, Wq, K_ext, V_ext, Wo):
    idx = lax.axis_index("i")
    Wq_sl = lax.dynamic_slice_in_dim(Wq, idx * DB, DB, axis=1).astype(jnp.bfloat16)
    Wo_sl = lax.dynamic_slice_in_dim(Wo, idx * DB, DB, axis=0).astype(jnp.bfloat16)
    Kf = K_ext.reshape(B, SKV, HL * DH)
    Vf = V_ext.reshape(B, SKV, HL * DH)

    def body(x_ref, wq_ref, k_ref, v_ref, wo_ref, out_ref,
             send_ref, recv_ref, send_sems, recv_sems):
        my = lax.axis_index("i")
        nbrs = [lax.bitwise_xor(my, c) for c in XOR_AXES]

        barrier = pltpu.get_barrier_semaphore()
        for nbr in nbrs:
            pl.semaphore_signal(barrier, inc=1, device_id=(nbr,),
                                device_id_type=pl.DeviceIdType.MESH)

        xb = x_ref[...].reshape(ROWS, DM).astype(jnp.bfloat16)
        q = lax.dot(xb, wq_ref[...],
                    preferred_element_type=jnp.float32).astype(jnp.bfloat16)
        ctx_rows = []
        for b in range(B):
            kb = k_ref[b].astype(jnp.bfloat16)
            vb = v_ref[b].astype(jnp.bfloat16)
            head_cols = []
            for h in range(HL):
                qbh = q[b * SQ:(b + 1) * SQ, h * DH:(h + 1) * DH]
                kbh = kb[:, h * DH:(h + 1) * DH]
                vbh = vb[:, h * DH:(h + 1) * DH]
                s = lax.dot_general(
                    qbh, kbh, (((1,), (1,)), ((), ())),
                    preferred_element_type=jnp.float32) * 0.125
                w = jnp.exp(s)
                recip = 1.0 / jnp.sum(w, axis=1, keepdims=True)
                ctx = lax.dot(w.astype(jnp.bfloat16), vbh,
                              preferred_element_type=jnp.float32)
                head_cols.append((ctx * recip).astype(jnp.bfloat16))
            ctx_rows.append(jnp.concatenate(head_cols, axis=1))
        ctx_all = jnp.concatenate(ctx_rows, axis=0)
        acc = lax.dot(ctx_all, wo_ref[...],
                      preferred_element_type=jnp.float32)

        pl.semaphore_wait(barrier, len(nbrs))
        for s in range(3):
            rdmas = []
            for p, (r0, rn) in enumerate(CHUNKS):
                send_ref[s, pl.ds(r0, rn), :] = acc[r0:r0 + rn, :].astype(jnp.bfloat16)
                rdma = pltpu.make_async_remote_copy(
                    src_ref=send_ref.at[s, pl.ds(r0, rn)],
                    dst_ref=recv_ref.at[s, pl.ds(r0, rn)],
                    send_sem=send_sems.at[s, p],
                    recv_sem=recv_sems.at[s, p],
                    device_id=(nbrs[(s + p) % 3],),
                    device_id_type=pl.DeviceIdType.MESH,
                )
                rdma.start()
                rdmas.append(rdma)
            parts = []
            for p, (r0, rn) in enumerate(CHUNKS):
                rdmas[p].wait()
                parts.append(acc[r0:r0 + rn, :]
                             + recv_ref[s, pl.ds(r0, rn), :].astype(jnp.float32))
            acc = jnp.concatenate(parts, axis=0)
        out_ref[...] = acc.reshape(B, SQ, DM).astype(jnp.bfloat16)

    return pl.pallas_call(
        body,
        out_shape=jax.ShapeDtypeStruct((B, SQ, DM), jnp.bfloat16),
        in_specs=[pl.BlockSpec(memory_space=pltpu.VMEM)] * 5,
        out_specs=pl.BlockSpec(memory_space=pltpu.VMEM),
        scratch_shapes=[
            pltpu.VMEM((3, ROWS, DM), jnp.bfloat16),
            pltpu.VMEM((3, ROWS, DM), jnp.bfloat16),
            pltpu.SemaphoreType.DMA((3, 3)),
            pltpu.SemaphoreType.DMA((3, 3)),
        ],
        compiler_params=pltpu.CompilerParams(collective_id=0),
    )(x, Wq_sl, Kf, Vf, Wo_sl)
